# baseline (speedup 1.0000x reference)
import functools

import jax
import jax.numpy as jnp
from jax import lax
from jax.experimental import pallas as pl
from jax.experimental.pallas import tpu as pltpu
from jax.experimental.pallas import tpu_sc as plsc

N = 10000
E = 320000
D_IN = 128
LATENT = 64
D_OUT = 128

NC = 2
NS = 16
EPT = E // NS
C = 100
NCH = EPT // C
RPT = 632
NPAD = NS * RPT
DW = 8

_MESH = plsc.VectorSubcoreMesh(core_axis_name="c", subcore_axis_name="s")


def _sc_aggregate(D, with_deg):
    def body(table2, src2, dst2, zacc, zdeg, ones_in, acc_out, deg_out,
             src_v, dst_v, rows0, rows1, acc_sh, sem0, sem1,
             ones_v=None, deg_sh=None):
        cid = lax.axis_index("c")
        sid = lax.axis_index("s")
        row0 = sid * RPT
        pltpu.sync_copy(zacc, acc_sh.at[pl.ds(row0, RPT)])
        if with_deg:
            @pl.when(cid == 0)
            def _():
                pltpu.sync_copy(zdeg, deg_sh.at[pl.ds(row0, RPT)])
                pltpu.sync_copy(ones_in, ones_v)
        pltpu.sync_copy(src2.at[sid], src_v)
        pltpu.sync_copy(dst2.at[sid], dst_v)
        plsc.subcore_barrier()

        table = table2.at[cid]

        pltpu.async_copy(table.at[src_v.at[0]], rows0, sem0)
        pltpu.async_copy(table.at[src_v.at[1]], rows1, sem1)

        def step(p, carry):
            j0 = 2 * p
            pltpu.make_async_copy(table.at[src_v.at[j0]], rows0, sem0).wait()
            pltpu.sync_copy(rows0, acc_sh.at[dst_v.at[j0]], add=True)
            if with_deg:
                @pl.when(cid == 0)
                def _():
                    pltpu.sync_copy(ones_v, deg_sh.at[dst_v.at[j0]], add=True)

            @pl.when(j0 + 2 < NCH)
            def _():
                pltpu.async_copy(table.at[src_v.at[j0 + 2]], rows0, sem0)

            pltpu.make_async_copy(table.at[src_v.at[j0 + 1]], rows1,
                                  sem1).wait()
            pltpu.sync_copy(rows1, acc_sh.at[dst_v.at[j0 + 1]], add=True)
            if with_deg:
                @pl.when(cid == 0)
                def _():
                    pltpu.sync_copy(ones_v, deg_sh.at[dst_v.at[j0 + 1]],
                                    add=True)

            @pl.when(j0 + 3 < NCH)
            def _():
                pltpu.async_copy(table.at[src_v.at[j0 + 3]], rows1, sem1)

            return carry

        lax.fori_loop(0, NCH // 2, step, 0)
        plsc.subcore_barrier()
        pltpu.sync_copy(acc_sh.at[pl.ds(row0, RPT)],
                        acc_out.at[cid].at[pl.ds(row0, RPT)])
        if with_deg:
            @pl.when(cid == 0)
            def _():
                pltpu.sync_copy(deg_sh.at[pl.ds(row0, RPT)],
                                deg_out.at[pl.ds(row0, RPT)])

    out_type = [jax.ShapeDtypeStruct((NC, NPAD, D), jnp.float32)]
    scratch = [
        pltpu.VMEM((NCH, C), jnp.int32),
        pltpu.VMEM((NCH, C), jnp.int32),
        pltpu.VMEM((C, D), jnp.float32),
        pltpu.VMEM((C, D), jnp.float32),
        pltpu.VMEM_SHARED((NPAD, D), jnp.float32),
        pltpu.SemaphoreType.DMA,
        pltpu.SemaphoreType.DMA,
    ]
    if with_deg:
        out_type.append(jax.ShapeDtypeStruct((NPAD, DW), jnp.float32))
        scratch += [
            pltpu.VMEM((C, DW), jnp.float32),
            pltpu.VMEM_SHARED((NPAD, DW), jnp.float32),
        ]
        fn = body
    else:
        def fn(table2, src2, dst2, zacc, acc_out, src_v, dst_v, rows0, rows1,
               acc_sh, sem0, sem1):
            body(table2, src2, dst2, zacc, None, None, acc_out, None,
                 src_v, dst_v, rows0, rows1, acc_sh, sem0, sem1)

    return pl.kernel(
        fn,
        out_type=tuple(out_type) if with_deg else out_type[0],
        mesh=_MESH,
        compiler_params=pltpu.CompilerParams(use_tc_tiling_on_sc=False),
        scratch_types=tuple(scratch),
    )


HD = D_IN // NC
HL = LATENT // NC

_sc_pass1 = _sc_aggregate(HD, with_deg=True)
_sc_pass2 = _sc_aggregate(HL, with_deg=False)

BN = 400
GRID = N // BN


def _enc_body(h_ref, a_ref, d_ref, n_ref, wsm, wsl, wnm0, wnm1, wnl0, wnl1,
              bm, bl, z_ref):
    inv = 1.0 / jnp.maximum(d_ref[:, 0:1], 1.0)
    a0 = a_ref[0] * inv
    a1 = a_ref[1] * inv
    hh = h_ref[...]
    dot = functools.partial(jnp.dot, preferred_element_type=jnp.float32)
    mean = dot(hh, wsm[...]) + dot(a0, wnm0[...]) + dot(a1, wnm1[...]) + bm[...]
    logvar = dot(hh, wsl[...]) + dot(a0, wnl0[...]) + dot(a1, wnl1[...]) + bl[...]
    logvar = jnp.clip(logvar, -30.0, 20.0)
    z_ref[...] = mean + jnp.exp(0.5 * logvar) * n_ref[...]


def _dec_body(z_ref, a_ref, d_ref, ws, wn0, wn1, b_ref, o_ref):
    inv = 1.0 / jnp.maximum(d_ref[:, 0:1], 1.0)
    a0 = a_ref[0] * inv
    a1 = a_ref[1] * inv
    dot = functools.partial(jnp.dot, preferred_element_type=jnp.float32)
    o_ref[...] = (dot(z_ref[...], ws[...]) + dot(a0, wn0[...])
                  + dot(a1, wn1[...]) + b_ref[...])


def _row_spec(w):
    return pl.BlockSpec((BN, w), lambda i: (i, 0))


def _part_spec(w):
    return pl.BlockSpec((NC, BN, w), lambda i: (0, i, 0))


def _full_spec(r, c):
    return pl.BlockSpec((r, c), lambda i: (0, 0))


_tc_encode = pl.pallas_call(
    _enc_body,
    grid=(GRID,),
    in_specs=[
        _row_spec(D_IN), _part_spec(HD), _row_spec(DW), _row_spec(LATENT),
        _full_spec(D_IN, LATENT), _full_spec(D_IN, LATENT),
        _full_spec(HD, LATENT), _full_spec(HD, LATENT),
        _full_spec(HD, LATENT), _full_spec(HD, LATENT),
        _full_spec(1, LATENT), _full_spec(1, LATENT),
    ],
    out_specs=_row_spec(LATENT),
    out_shape=jax.ShapeDtypeStruct((N, LATENT), jnp.float32),
)

_tc_decode = pl.pallas_call(
    _dec_body,
    grid=(GRID,),
    in_specs=[
        _row_spec(LATENT), _part_spec(HL), _row_spec(DW),
        _full_spec(LATENT, D_OUT), _full_spec(HL, D_OUT),
        _full_spec(HL, D_OUT), _full_spec(1, D_OUT),
    ],
    out_specs=_row_spec(D_OUT),
    out_shape=jax.ShapeDtypeStruct((N, D_OUT), jnp.float32),
)


def kernel(x, edge_index, noise, W_enc_self, W_enc_nbr, b_enc,
           W_dec_self, W_dec_nbr, b_dec):
    b = x.shape[0]
    h = x.reshape(N, D_IN)
    h2 = jnp.stack([h[:, :HD], h[:, HD:]])
    src2 = edge_index[0].reshape(NS, NCH, C)
    dst2 = edge_index[1].reshape(NS, NCH, C)
    zacc1 = jnp.zeros((RPT, HD), jnp.float32)
    zacc2 = jnp.zeros((RPT, HL), jnp.float32)
    zdeg = jnp.zeros((RPT, DW), jnp.float32)
    ones = jnp.ones((C, DW), jnp.float32)

    agg_parts, deg = _sc_pass1(h2, src2, dst2, zacc1, zdeg, ones)
    z = _tc_encode(
        h, agg_parts, deg, noise.reshape(N, LATENT),
        W_enc_self[:, :LATENT], W_enc_self[:, LATENT:],
        W_enc_nbr[:HD, :LATENT], W_enc_nbr[HD:, :LATENT],
        W_enc_nbr[:HD, LATENT:], W_enc_nbr[HD:, LATENT:],
        b_enc[:LATENT].reshape(1, LATENT), b_enc[LATENT:].reshape(1, LATENT),
    )
    z2 = jnp.stack([z[:, :HL], z[:, HL:]])
    zagg_parts = _sc_pass2(z2, src2, dst2, zacc2)
    dec = _tc_decode(
        z, zagg_parts, deg,
        W_dec_self, W_dec_nbr[:HL], W_dec_nbr[HL:], b_dec.reshape(1, D_OUT),
    )
    return dec.reshape(b, N, D_OUT)


# speedup vs baseline: 1.2604x; 1.2604x over previous
import functools

import jax
import jax.numpy as jnp
from jax import lax
from jax.experimental import pallas as pl
from jax.experimental.pallas import tpu as pltpu
from jax.experimental.pallas import tpu_sc as plsc

N = 10000
E = 320000
D_IN = 128
LATENT = 64
D_OUT = 128

NC = 2
NS = 16
NW = NC * NS
EPW = E // NW
C = 80
NCH = EPW // C
RPT = 632
NPAD = NS * RPT
DW = 8

_MESH = plsc.VectorSubcoreMesh(core_axis_name="c", subcore_axis_name="s")


def _sc_aggregate(D, with_deg):
    def body(table, src3, dst3, zacc, zdeg, ones_in, acc_out, deg_out,
             src_v, dst_v, rows0, rows1, acc_sh, sem0, sem1,
             ones_v=None, deg_sh=None):
        cid = lax.axis_index("c")
        sid = lax.axis_index("s")
        wid = cid * NS + sid
        row0 = sid * RPT
        pltpu.sync_copy(zacc, acc_sh.at[pl.ds(row0, RPT)])
        if with_deg:
            pltpu.sync_copy(zdeg, deg_sh.at[pl.ds(row0, RPT)])
            pltpu.sync_copy(ones_in, ones_v)
        pltpu.sync_copy(src3.at[wid], src_v)
        pltpu.sync_copy(dst3.at[wid], dst_v)
        plsc.subcore_barrier()

        pltpu.async_copy(table.at[src_v.at[0]], rows0, sem0)
        pltpu.async_copy(table.at[src_v.at[1]], rows1, sem1)

        def step(p, carry):
            j0 = 2 * p
            pltpu.make_async_copy(table.at[src_v.at[j0]], rows0, sem0).wait()
            pltpu.sync_copy(rows0, acc_sh.at[dst_v.at[j0]], add=True)
            if with_deg:
                pltpu.sync_copy(ones_v, deg_sh.at[dst_v.at[j0]], add=True)

            @pl.when(j0 + 2 < NCH)
            def _():
                pltpu.async_copy(table.at[src_v.at[j0 + 2]], rows0, sem0)

            pltpu.make_async_copy(table.at[src_v.at[j0 + 1]], rows1,
                                  sem1).wait()
            pltpu.sync_copy(rows1, acc_sh.at[dst_v.at[j0 + 1]], add=True)
            if with_deg:
                pltpu.sync_copy(ones_v, deg_sh.at[dst_v.at[j0 + 1]], add=True)

            @pl.when(j0 + 3 < NCH)
            def _():
                pltpu.async_copy(table.at[src_v.at[j0 + 3]], rows1, sem1)

            return carry

        lax.fori_loop(0, NCH // 2, step, 0)
        if NCH % 2:
            pltpu.make_async_copy(table.at[src_v.at[NCH - 1]], rows0,
                                  sem0).wait()
            pltpu.sync_copy(rows0, acc_sh.at[dst_v.at[NCH - 1]], add=True)
            if with_deg:
                pltpu.sync_copy(ones_v, deg_sh.at[dst_v.at[NCH - 1]],
                                add=True)
        plsc.subcore_barrier()
        pltpu.sync_copy(acc_sh.at[pl.ds(row0, RPT)],
                        acc_out.at[cid].at[pl.ds(row0, RPT)])
        if with_deg:
            pltpu.sync_copy(deg_sh.at[pl.ds(row0, RPT)],
                            deg_out.at[cid].at[pl.ds(row0, RPT)])

    out_type = [jax.ShapeDtypeStruct((NC, NPAD, D), jnp.float32)]
    scratch = [
        pltpu.VMEM((NCH, C), jnp.int32),
        pltpu.VMEM((NCH, C), jnp.int32),
        pltpu.VMEM((C, D), jnp.float32),
        pltpu.VMEM((C, D), jnp.float32),
        pltpu.VMEM_SHARED((NPAD, D), jnp.float32),
        pltpu.SemaphoreType.DMA,
        pltpu.SemaphoreType.DMA,
    ]
    if with_deg:
        out_type.append(jax.ShapeDtypeStruct((NC, NPAD, DW), jnp.float32))
        scratch += [
            pltpu.VMEM((C, DW), jnp.float32),
            pltpu.VMEM_SHARED((NPAD, DW), jnp.float32),
        ]
        fn = body
    else:
        def fn(table, src3, dst3, zacc, acc_out, src_v, dst_v, rows0, rows1,
               acc_sh, sem0, sem1):
            body(table, src3, dst3, zacc, None, None, acc_out, None,
                 src_v, dst_v, rows0, rows1, acc_sh, sem0, sem1)

    return pl.kernel(
        fn,
        out_type=tuple(out_type) if with_deg else out_type[0],
        mesh=_MESH,
        compiler_params=pltpu.CompilerParams(use_tc_tiling_on_sc=False),
        scratch_types=tuple(scratch),
    )


_sc_pass1 = _sc_aggregate(D_IN, with_deg=True)
_sc_pass2 = _sc_aggregate(LATENT, with_deg=False)

BN = 400
GRID = N // BN


def _enc_body(h_ref, a_ref, d_ref, n_ref, wsm, wsl, wnm, wnl, bm, bl, z_ref):
    deg = jnp.maximum(d_ref[0, :, 0:1] + d_ref[1, :, 0:1], 1.0)
    agg = (a_ref[0] + a_ref[1]) / deg
    hh = h_ref[...]
    dot = functools.partial(jnp.dot, preferred_element_type=jnp.float32)
    mean = dot(hh, wsm[...]) + dot(agg, wnm[...]) + bm[...]
    logvar = dot(hh, wsl[...]) + dot(agg, wnl[...]) + bl[...]
    logvar = jnp.clip(logvar, -30.0, 20.0)
    z_ref[...] = mean + jnp.exp(0.5 * logvar) * n_ref[...]


def _dec_body(z_ref, a_ref, d_ref, ws, wn, b_ref, o_ref):
    deg = jnp.maximum(d_ref[0, :, 0:1] + d_ref[1, :, 0:1], 1.0)
    zagg = (a_ref[0] + a_ref[1]) / deg
    dot = functools.partial(jnp.dot, preferred_element_type=jnp.float32)
    o_ref[...] = dot(z_ref[...], ws[...]) + dot(zagg, wn[...]) + b_ref[...]


def _row_spec(w):
    return pl.BlockSpec((BN, w), lambda i: (i, 0))


def _part_spec(w):
    return pl.BlockSpec((NC, BN, w), lambda i: (0, i, 0))


def _full_spec(r, c):
    return pl.BlockSpec((r, c), lambda i: (0, 0))


_tc_encode = pl.pallas_call(
    _enc_body,
    grid=(GRID,),
    in_specs=[
        _row_spec(D_IN), _part_spec(D_IN), _part_spec(DW), _row_spec(LATENT),
        _full_spec(D_IN, LATENT), _full_spec(D_IN, LATENT),
        _full_spec(D_IN, LATENT), _full_spec(D_IN, LATENT),
        _full_spec(1, LATENT), _full_spec(1, LATENT),
    ],
    out_specs=_row_spec(LATENT),
    out_shape=jax.ShapeDtypeStruct((N, LATENT), jnp.float32),
)

_tc_decode = pl.pallas_call(
    _dec_body,
    grid=(GRID,),
    in_specs=[
        _row_spec(LATENT), _part_spec(LATENT), _part_spec(DW),
        _full_spec(LATENT, D_OUT), _full_spec(LATENT, D_OUT),
        _full_spec(1, D_OUT),
    ],
    out_specs=_row_spec(D_OUT),
    out_shape=jax.ShapeDtypeStruct((N, D_OUT), jnp.float32),
)


def kernel(x, edge_index, noise, W_enc_self, W_enc_nbr, b_enc,
           W_dec_self, W_dec_nbr, b_dec):
    b = x.shape[0]
    h = x.reshape(N, D_IN)
    src3 = edge_index[0].reshape(NW, NCH, C)
    dst3 = edge_index[1].reshape(NW, NCH, C)
    zacc1 = jnp.zeros((RPT, D_IN), jnp.float32)
    zacc2 = jnp.zeros((RPT, LATENT), jnp.float32)
    zdeg = jnp.zeros((RPT, DW), jnp.float32)
    ones = jnp.ones((C, DW), jnp.float32)

    agg_parts, deg_parts = _sc_pass1(h, src3, dst3, zacc1, zdeg, ones)
    z = _tc_encode(
        h, agg_parts, deg_parts, noise.reshape(N, LATENT),
        W_enc_self[:, :LATENT], W_enc_self[:, LATENT:],
        W_enc_nbr[:, :LATENT], W_enc_nbr[:, LATENT:],
        b_enc[:LATENT].reshape(1, LATENT), b_enc[LATENT:].reshape(1, LATENT),
    )
    zagg_parts = _sc_pass2(z, src3, dst3, zacc2)
    dec = _tc_decode(
        z, zagg_parts, deg_parts,
        W_dec_self, W_dec_nbr, b_dec.reshape(1, D_OUT),
    )
    return dec.reshape(b, N, D_OUT)


# speedup vs baseline: 1.2913x; 1.0246x over previous
import functools

import jax
import jax.numpy as jnp
from jax import lax
from jax.experimental import pallas as pl
from jax.experimental.pallas import tpu as pltpu
from jax.experimental.pallas import tpu_sc as plsc

N = 10000
E = 320000
D_IN = 128
LATENT = 64
D_OUT = 128

NC = 2
NS = 16
NW = NC * NS
EPW = E // NW
C1 = 80
C2 = 100
RPT = 632
NPAD = NS * RPT
DW = 8

_MESH = plsc.VectorSubcoreMesh(core_axis_name="c", subcore_axis_name="s")


def _sc_aggregate(D, with_deg, C):
    NCH = EPW // C

    def body(table, src3, dst3, zacc, zdeg, ones_in, acc_out, deg_out,
             src_v, dst_v, rows0, rows1, acc_sh, sem0, sem1,
             ones_v=None, deg_sh=None):
        cid = lax.axis_index("c")
        sid = lax.axis_index("s")
        wid = cid * NS + sid
        row0 = sid * RPT
        pltpu.sync_copy(zacc, acc_sh.at[pl.ds(row0, RPT)])
        if with_deg:
            pltpu.sync_copy(zdeg, deg_sh.at[pl.ds(row0, RPT)])
            pltpu.sync_copy(ones_in, ones_v)
        pltpu.sync_copy(src3.at[wid], src_v)
        pltpu.sync_copy(dst3.at[wid], dst_v)
        plsc.subcore_barrier()

        pltpu.async_copy(table.at[src_v.at[0]], rows0, sem0)
        pltpu.async_copy(table.at[src_v.at[1]], rows1, sem1)

        def step(p, carry):
            j0 = 2 * p
            pltpu.make_async_copy(table.at[src_v.at[j0]], rows0, sem0).wait()
            pltpu.sync_copy(rows0, acc_sh.at[dst_v.at[j0]], add=True)
            if with_deg:
                pltpu.sync_copy(ones_v, deg_sh.at[dst_v.at[j0]], add=True)

            @pl.when(j0 + 2 < NCH)
            def _():
                pltpu.async_copy(table.at[src_v.at[j0 + 2]], rows0, sem0)

            pltpu.make_async_copy(table.at[src_v.at[j0 + 1]], rows1,
                                  sem1).wait()
            pltpu.sync_copy(rows1, acc_sh.at[dst_v.at[j0 + 1]], add=True)
            if with_deg:
                pltpu.sync_copy(ones_v, deg_sh.at[dst_v.at[j0 + 1]], add=True)

            @pl.when(j0 + 3 < NCH)
            def _():
                pltpu.async_copy(table.at[src_v.at[j0 + 3]], rows1, sem1)

            return carry

        lax.fori_loop(0, NCH // 2, step, 0)
        if NCH % 2:
            pltpu.make_async_copy(table.at[src_v.at[NCH - 1]], rows0,
                                  sem0).wait()
            pltpu.sync_copy(rows0, acc_sh.at[dst_v.at[NCH - 1]], add=True)
            if with_deg:
                pltpu.sync_copy(ones_v, deg_sh.at[dst_v.at[NCH - 1]],
                                add=True)
        plsc.subcore_barrier()
        pltpu.sync_copy(acc_sh.at[pl.ds(row0, RPT)],
                        acc_out.at[cid].at[pl.ds(row0, RPT)])
        if with_deg:
            pltpu.sync_copy(deg_sh.at[pl.ds(row0, RPT)],
                            deg_out.at[cid].at[pl.ds(row0, RPT)])

    out_type = [jax.ShapeDtypeStruct((NC, NPAD, D), jnp.float32)]
    scratch = [
        pltpu.VMEM((NCH, C), jnp.int32),
        pltpu.VMEM((NCH, C), jnp.int32),
        pltpu.VMEM((C, D), jnp.float32),
        pltpu.VMEM((C, D), jnp.float32),
        pltpu.VMEM_SHARED((NPAD, D), jnp.float32),
        pltpu.SemaphoreType.DMA,
        pltpu.SemaphoreType.DMA,
    ]
    if with_deg:
        out_type.append(jax.ShapeDtypeStruct((NC, NPAD, DW), jnp.float32))
        scratch += [
            pltpu.VMEM((C, DW), jnp.float32),
            pltpu.VMEM_SHARED((NPAD, DW), jnp.float32),
        ]
        fn = body
    else:
        def fn(table, src3, dst3, zacc, acc_out, src_v, dst_v, rows0, rows1,
               acc_sh, sem0, sem1):
            body(table, src3, dst3, zacc, None, None, acc_out, None,
                 src_v, dst_v, rows0, rows1, acc_sh, sem0, sem1)

    return pl.kernel(
        fn,
        out_type=tuple(out_type) if with_deg else out_type[0],
        mesh=_MESH,
        compiler_params=pltpu.CompilerParams(use_tc_tiling_on_sc=False),
        scratch_types=tuple(scratch),
    )


_sc_pass1 = _sc_aggregate(D_IN, with_deg=True, C=C1)
_sc_pass2 = _sc_aggregate(LATENT, with_deg=False, C=C2)

BN = 400
GRID = N // BN


def _enc_body(h_ref, a_ref, d_ref, n_ref, wsm, wsl, wnm, wnl, bm, bl, z_ref):
    deg = jnp.maximum(d_ref[0, :, 0:1] + d_ref[1, :, 0:1], 1.0)
    agg = (a_ref[0] + a_ref[1]) / deg
    hh = h_ref[...]
    dot = functools.partial(jnp.dot, preferred_element_type=jnp.float32)
    mean = dot(hh, wsm[...]) + dot(agg, wnm[...]) + bm[...]
    logvar = dot(hh, wsl[...]) + dot(agg, wnl[...]) + bl[...]
    logvar = jnp.clip(logvar, -30.0, 20.0)
    z_ref[...] = mean + jnp.exp(0.5 * logvar) * n_ref[...]


def _dec_body(z_ref, a_ref, d_ref, ws, wn, b_ref, o_ref):
    deg = jnp.maximum(d_ref[0, :, 0:1] + d_ref[1, :, 0:1], 1.0)
    zagg = (a_ref[0] + a_ref[1]) / deg
    dot = functools.partial(jnp.dot, preferred_element_type=jnp.float32)
    o_ref[...] = dot(z_ref[...], ws[...]) + dot(zagg, wn[...]) + b_ref[...]


def _row_spec(w):
    return pl.BlockSpec((BN, w), lambda i: (i, 0))


def _part_spec(w):
    return pl.BlockSpec((NC, BN, w), lambda i: (0, i, 0))


def _full_spec(r, c):
    return pl.BlockSpec((r, c), lambda i: (0, 0))


_tc_encode = pl.pallas_call(
    _enc_body,
    grid=(GRID,),
    in_specs=[
        _row_spec(D_IN), _part_spec(D_IN), _part_spec(DW), _row_spec(LATENT),
        _full_spec(D_IN, LATENT), _full_spec(D_IN, LATENT),
        _full_spec(D_IN, LATENT), _full_spec(D_IN, LATENT),
        _full_spec(1, LATENT), _full_spec(1, LATENT),
    ],
    out_specs=_row_spec(LATENT),
    out_shape=jax.ShapeDtypeStruct((N, LATENT), jnp.float32),
)

_tc_decode = pl.pallas_call(
    _dec_body,
    grid=(GRID,),
    in_specs=[
        _row_spec(LATENT), _part_spec(LATENT), _part_spec(DW),
        _full_spec(LATENT, D_OUT), _full_spec(LATENT, D_OUT),
        _full_spec(1, D_OUT),
    ],
    out_specs=_row_spec(D_OUT),
    out_shape=jax.ShapeDtypeStruct((N, D_OUT), jnp.float32),
)


def kernel(x, edge_index, noise, W_enc_self, W_enc_nbr, b_enc,
           W_dec_self, W_dec_nbr, b_dec):
    b = x.shape[0]
    h = x.reshape(N, D_IN)
    src1 = edge_index[0].reshape(NW, EPW // C1, C1)
    dst1 = edge_index[1].reshape(NW, EPW // C1, C1)
    src2 = edge_index[0].reshape(NW, EPW // C2, C2)
    dst2 = edge_index[1].reshape(NW, EPW // C2, C2)
    zacc1 = jnp.zeros((RPT, D_IN), jnp.float32)
    zacc2 = jnp.zeros((RPT, LATENT), jnp.float32)
    zdeg = jnp.zeros((RPT, DW), jnp.float32)
    ones = jnp.ones((C1, DW), jnp.float32)

    agg_parts, deg_parts = _sc_pass1(h, src1, dst1, zacc1, zdeg, ones)
    z = _tc_encode(
        h, agg_parts, deg_parts, noise.reshape(N, LATENT),
        W_enc_self[:, :LATENT], W_enc_self[:, LATENT:],
        W_enc_nbr[:, :LATENT], W_enc_nbr[:, LATENT:],
        b_enc[:LATENT].reshape(1, LATENT), b_enc[LATENT:].reshape(1, LATENT),
    )
    zagg_parts = _sc_pass2(z, src2, dst2, zacc2)
    dec = _tc_decode(
        z, zagg_parts, deg_parts,
        W_dec_self, W_dec_nbr, b_dec.reshape(1, D_OUT),
    )
    return dec.reshape(b, N, D_OUT)


# speedup vs baseline: 1.3286x; 1.0289x over previous
import functools

import jax
import jax.numpy as jnp
from jax import lax
from jax.experimental import pallas as pl
from jax.experimental.pallas import tpu as pltpu
from jax.experimental.pallas import tpu_sc as plsc

N = 10000
E = 320000
D_IN = 128
LATENT = 64
D_OUT = 128

NC = 2
NS = 16
NW = NC * NS
EPW = E // NW
C1 = 100
C2 = 125
RPT = 632
NPAD = NS * RPT
DW = 8

_MESH = plsc.VectorSubcoreMesh(core_axis_name="c", subcore_axis_name="s")


def _sc_aggregate(D, with_deg, C):
    NCH = EPW // C

    def body(table, src3, dst3, zacc, zdeg, ones_in, acc_out, deg_out,
             src_v, dst_v, rows0, rows1, acc_sh, sem0, sem1,
             ones_v=None, deg_sh=None):
        cid = lax.axis_index("c")
        sid = lax.axis_index("s")
        wid = cid * NS + sid
        row0 = sid * RPT
        pltpu.sync_copy(zacc, acc_sh.at[pl.ds(row0, RPT)])
        if with_deg:
            pltpu.sync_copy(zdeg, deg_sh.at[pl.ds(row0, RPT)])
            pltpu.sync_copy(ones_in, ones_v)
        pltpu.sync_copy(src3.at[wid], src_v)
        pltpu.sync_copy(dst3.at[wid], dst_v)
        plsc.subcore_barrier()

        pltpu.async_copy(table.at[src_v.at[0]], rows0, sem0)
        pltpu.async_copy(table.at[src_v.at[1]], rows1, sem1)

        def step(p, carry):
            j0 = 2 * p
            pltpu.make_async_copy(table.at[src_v.at[j0]], rows0, sem0).wait()
            pltpu.sync_copy(rows0, acc_sh.at[dst_v.at[j0]], add=True)
            if with_deg:
                pltpu.sync_copy(ones_v, deg_sh.at[dst_v.at[j0]], add=True)

            @pl.when(j0 + 2 < NCH)
            def _():
                pltpu.async_copy(table.at[src_v.at[j0 + 2]], rows0, sem0)

            pltpu.make_async_copy(table.at[src_v.at[j0 + 1]], rows1,
                                  sem1).wait()
            pltpu.sync_copy(rows1, acc_sh.at[dst_v.at[j0 + 1]], add=True)
            if with_deg:
                pltpu.sync_copy(ones_v, deg_sh.at[dst_v.at[j0 + 1]], add=True)

            @pl.when(j0 + 3 < NCH)
            def _():
                pltpu.async_copy(table.at[src_v.at[j0 + 3]], rows1, sem1)

            return carry

        lax.fori_loop(0, NCH // 2, step, 0)
        if NCH % 2:
            pltpu.make_async_copy(table.at[src_v.at[NCH - 1]], rows0,
                                  sem0).wait()
            pltpu.sync_copy(rows0, acc_sh.at[dst_v.at[NCH - 1]], add=True)
            if with_deg:
                pltpu.sync_copy(ones_v, deg_sh.at[dst_v.at[NCH - 1]],
                                add=True)
        plsc.subcore_barrier()
        pltpu.sync_copy(acc_sh.at[pl.ds(row0, RPT)],
                        acc_out.at[cid].at[pl.ds(row0, RPT)])
        if with_deg:
            pltpu.sync_copy(deg_sh.at[pl.ds(row0, RPT)],
                            deg_out.at[cid].at[pl.ds(row0, RPT)])

    out_type = [jax.ShapeDtypeStruct((NC, NPAD, D), jnp.float32)]
    scratch = [
        pltpu.VMEM((NCH, C), jnp.int32),
        pltpu.VMEM((NCH, C), jnp.int32),
        pltpu.VMEM((C, D), jnp.float32),
        pltpu.VMEM((C, D), jnp.float32),
        pltpu.VMEM_SHARED((NPAD, D), jnp.float32),
        pltpu.SemaphoreType.DMA,
        pltpu.SemaphoreType.DMA,
    ]
    if with_deg:
        out_type.append(jax.ShapeDtypeStruct((NC, NPAD, DW), jnp.float32))
        scratch += [
            pltpu.VMEM((C, DW), jnp.float32),
            pltpu.VMEM_SHARED((NPAD, DW), jnp.float32),
        ]
        fn = body
    else:
        def fn(table, src3, dst3, zacc, acc_out, src_v, dst_v, rows0, rows1,
               acc_sh, sem0, sem1):
            body(table, src3, dst3, zacc, None, None, acc_out, None,
                 src_v, dst_v, rows0, rows1, acc_sh, sem0, sem1)

    return pl.kernel(
        fn,
        out_type=tuple(out_type) if with_deg else out_type[0],
        mesh=_MESH,
        compiler_params=pltpu.CompilerParams(use_tc_tiling_on_sc=False),
        scratch_types=tuple(scratch),
    )


def _sc_degree(C):
    NCH = EPW // C

    def body(dst3, zdeg, ones_in, deg_out, dst_v, ones_v, deg_sh):
        cid = lax.axis_index("c")
        sid = lax.axis_index("s")
        wid = cid * NS + sid
        row0 = sid * RPT
        pltpu.sync_copy(zdeg, deg_sh.at[pl.ds(row0, RPT)])
        pltpu.sync_copy(ones_in, ones_v)
        pltpu.sync_copy(dst3.at[wid], dst_v)
        plsc.subcore_barrier()

        def step(j, carry):
            pltpu.sync_copy(ones_v, deg_sh.at[dst_v.at[j]], add=True)
            return carry

        lax.fori_loop(0, NCH, step, 0)
        plsc.subcore_barrier()
        pltpu.sync_copy(deg_sh.at[pl.ds(row0, RPT)],
                        deg_out.at[cid].at[pl.ds(row0, RPT)])

    return pl.kernel(
        body,
        out_type=jax.ShapeDtypeStruct((NC, NPAD, DW), jnp.float32),
        mesh=_MESH,
        compiler_params=pltpu.CompilerParams(use_tc_tiling_on_sc=False),
        scratch_types=(
            pltpu.VMEM((NCH, C), jnp.int32),
            pltpu.VMEM((C, DW), jnp.float32),
            pltpu.VMEM_SHARED((NPAD, DW), jnp.float32),
        ),
    )


_sc_pass1 = _sc_aggregate(D_IN, with_deg=False, C=C1)
_sc_deg = _sc_degree(C=C2)
_sc_pass2 = _sc_aggregate(LATENT, with_deg=False, C=C2)


BN = 400
GRID = N // BN


def _enc_body(h_ref, a_ref, d_ref, n_ref, wsm, wsl, wnm, wnl, bm, bl, z_ref):
    deg = jnp.maximum(d_ref[0, :, 0:1] + d_ref[1, :, 0:1], 1.0)
    agg = (a_ref[0] + a_ref[1]) / deg
    hh = h_ref[...]
    dot = functools.partial(jnp.dot, preferred_element_type=jnp.float32)
    mean = dot(hh, wsm[...]) + dot(agg, wnm[...]) + bm[...]
    logvar = dot(hh, wsl[...]) + dot(agg, wnl[...]) + bl[...]
    logvar = jnp.clip(logvar, -30.0, 20.0)
    z_ref[...] = mean + jnp.exp(0.5 * logvar) * n_ref[...]


def _dec_body(z_ref, a_ref, d_ref, ws, wn, b_ref, o_ref):
    deg = jnp.maximum(d_ref[0, :, 0:1] + d_ref[1, :, 0:1], 1.0)
    zagg = (a_ref[0] + a_ref[1]) / deg
    dot = functools.partial(jnp.dot, preferred_element_type=jnp.float32)
    o_ref[...] = dot(z_ref[...], ws[...]) + dot(zagg, wn[...]) + b_ref[...]


def _row_spec(w):
    return pl.BlockSpec((BN, w), lambda i: (i, 0))


def _part_spec(w):
    return pl.BlockSpec((NC, BN, w), lambda i: (0, i, 0))


def _full_spec(r, c):
    return pl.BlockSpec((r, c), lambda i: (0, 0))


_tc_encode = pl.pallas_call(
    _enc_body,
    grid=(GRID,),
    in_specs=[
        _row_spec(D_IN), _part_spec(D_IN), _part_spec(DW), _row_spec(LATENT),
        _full_spec(D_IN, LATENT), _full_spec(D_IN, LATENT),
        _full_spec(D_IN, LATENT), _full_spec(D_IN, LATENT),
        _full_spec(1, LATENT), _full_spec(1, LATENT),
    ],
    out_specs=_row_spec(LATENT),
    out_shape=jax.ShapeDtypeStruct((N, LATENT), jnp.float32),
)

_tc_decode = pl.pallas_call(
    _dec_body,
    grid=(GRID,),
    in_specs=[
        _row_spec(LATENT), _part_spec(LATENT), _part_spec(DW),
        _full_spec(LATENT, D_OUT), _full_spec(LATENT, D_OUT),
        _full_spec(1, D_OUT),
    ],
    out_specs=_row_spec(D_OUT),
    out_shape=jax.ShapeDtypeStruct((N, D_OUT), jnp.float32),
)


def kernel(x, edge_index, noise, W_enc_self, W_enc_nbr, b_enc,
           W_dec_self, W_dec_nbr, b_dec):
    b = x.shape[0]
    h = x.reshape(N, D_IN)
    src1 = edge_index[0].reshape(NW, EPW // C1, C1)
    dst1 = edge_index[1].reshape(NW, EPW // C1, C1)
    src2 = edge_index[0].reshape(NW, EPW // C2, C2)
    dst2 = edge_index[1].reshape(NW, EPW // C2, C2)
    zacc1 = jnp.zeros((RPT, D_IN), jnp.float32)
    zacc2 = jnp.zeros((RPT, LATENT), jnp.float32)
    zdeg = jnp.zeros((RPT, DW), jnp.float32)
    ones = jnp.ones((C2, DW), jnp.float32)

    deg_parts = _sc_deg(dst2, zdeg, ones)
    agg_parts = _sc_pass1(h, src1, dst1, zacc1)
    z = _tc_encode(
        h, agg_parts, deg_parts, noise.reshape(N, LATENT),
        W_enc_self[:, :LATENT], W_enc_self[:, LATENT:],
        W_enc_nbr[:, :LATENT], W_enc_nbr[:, LATENT:],
        b_enc[:LATENT].reshape(1, LATENT), b_enc[LATENT:].reshape(1, LATENT),
    )
    zagg_parts = _sc_pass2(z, src2, dst2, zacc2)
    dec = _tc_decode(
        z, zagg_parts, deg_parts,
        W_dec_self, W_dec_nbr, b_dec.reshape(1, D_OUT),
    )
    return dec.reshape(b, N, D_OUT)
